# initial kernel scaffold (unmeasured)
import functools

import jax
import jax.numpy as jnp
from jax import lax
from jax.experimental import pallas as pl
from jax.experimental.pallas import tpu as pltpu

N_DEV = 32


def _ar_body(p_ref, out_ref, acc_ref, comm_ref, send_sem, recv_sem, credit_sem):
    my = lax.axis_index("i")
    left = (my - 1) % N_DEV
    right = (my + 1) % N_DEV

    barrier = pltpu.get_barrier_semaphore()
    for nbr in (left, right):
        pl.semaphore_signal(
            barrier, inc=1, device_id=(nbr,),
            device_id_type=pl.DeviceIdType.MESH,
        )
    pl.semaphore_wait(barrier, 2)

    acc_ref[...] = p_ref[...]

    for k in range(N_DEV - 1):
        slot = k % 2
        s_idx = (my - k) % N_DEV
        r_idx = (my - k - 1) % N_DEV
        if k >= 2:
            pl.semaphore_wait(credit_sem.at[slot], 1)
        rdma = pltpu.make_async_remote_copy(
            src_ref=acc_ref.at[s_idx],
            dst_ref=comm_ref.at[slot],
            send_sem=send_sem.at[slot],
            recv_sem=recv_sem.at[slot],
            device_id=(right,),
            device_id_type=pl.DeviceIdType.MESH,
        )
        rdma.start()
        rdma.wait()
        acc_ref[r_idx] = acc_ref[r_idx] + comm_ref[slot]
        pl.semaphore_signal(
            credit_sem.at[slot], inc=1, device_id=(left,),
            device_id_type=pl.DeviceIdType.MESH,
        )

    own = (my + 1) % N_DEV
    out_ref[own] = acc_ref[own].astype(out_ref.dtype)

    for g in range(N_DEV - 1):
        t = (N_DEV - 1) + g
        slot = t % 2
        s_idx = (my + 1 - g) % N_DEV
        pl.semaphore_wait(credit_sem.at[slot], 1)
        rdma = pltpu.make_async_remote_copy(
            src_ref=out_ref.at[s_idx],
            dst_ref=out_ref.at[s_idx],
            send_sem=send_sem.at[slot],
            recv_sem=recv_sem.at[slot],
            device_id=(right,),
            device_id_type=pl.DeviceIdType.MESH,
        )
        rdma.start()
        rdma.wait()
        pl.semaphore_signal(
            credit_sem.at[slot], inc=1, device_id=(left,),
            device_id_type=pl.DeviceIdType.MESH,
        )


def _ring_allreduce(partial, cid):
    B, S, D = partial.shape
    rows = B * S
    c = rows // N_DEV
    p = partial.reshape(N_DEV, c, D)
    out = pl.pallas_call(
        _ar_body,
        out_shape=jax.ShapeDtypeStruct((N_DEV, c, D), jnp.bfloat16),
        in_specs=[pl.BlockSpec(memory_space=pltpu.VMEM)],
        out_specs=pl.BlockSpec(memory_space=pltpu.VMEM),
        scratch_shapes=[
            pltpu.VMEM((N_DEV, c, D), jnp.float32),
            pltpu.VMEM((2, c, D), jnp.float32),
            pltpu.SemaphoreType.DMA((2,)),
            pltpu.SemaphoreType.DMA((2,)),
            pltpu.SemaphoreType.REGULAR((2,)),
        ],
        compiler_params=pltpu.CompilerParams(collective_id=cid),
    )(p)
    return out.reshape(B, S, D)


def kernel(x, Wq, Wk, Wv, Wo, t_emb, W_mod, W_ff1, W_ff2):
    f32 = jnp.float32
    bf16 = jnp.bfloat16
    B, S, D = x.shape
    Dh = 128
    H = Wq.shape[1] // Dh

    mod = t_emb @ W_mod
    sa, sha, ga, sm, shm, gm = jnp.split(mod, 6, axis=-1)

    def ln(h):
        m = h.mean(axis=-1, keepdims=True)
        v = h.var(axis=-1, keepdims=True)
        return (h - m) * lax.rsqrt(v + 1e-5)

    x0 = x
    xm = (ln(x0) * (1.0 + sa[:, None, :]) + sha[:, None, :]).astype(bf16)

    Q = (xm @ Wq.astype(bf16)).reshape(B, S, H, Dh)
    K = (xm @ Wk.astype(bf16)).reshape(B, S, H, Dh)
    V = (xm @ Wv.astype(bf16)).reshape(B, S, H, Dh)
    scores = jnp.einsum(
        "bihd,bjhd->bhij", Q, K, preferred_element_type=f32
    ) * 0.08838834764831843
    P = jax.nn.softmax(scores, axis=-1).astype(bf16)
    O = jnp.einsum("bhij,bjhd->bihd", P, V, preferred_element_type=f32)
    attn_partial = O.reshape(B, S, H * Dh).astype(bf16) @ Wo.astype(bf16)
    attn_partial = attn_partial.astype(f32)

    attn_sum = _ring_allreduce(attn_partial, cid=0).astype(f32)
    x1 = x0 + ga[:, None, :] * attn_sum

    xm2 = (ln(x1) * (1.0 + sm[:, None, :]) + shm[:, None, :]).astype(bf16)
    h = (xm2 @ W_ff1.astype(bf16)).astype(f32)
    h = h * jax.nn.sigmoid(h)
    ff_partial = (h.astype(bf16) @ W_ff2.astype(bf16)).astype(f32)

    ff_sum = _ring_allreduce(ff_partial, cid=1).astype(f32)
    return x1 + gm[:, None, :] * ff_sum


# baseline (device time: 672607 ns/iter reference)
import functools

import jax
import jax.numpy as jnp
from jax import lax
from jax.experimental import pallas as pl
from jax.experimental.pallas import tpu as pltpu

N_DEV = 32


def _ar_body(p_ref, out_ref, acc_ref, comm_ref, send_sem, recv_sem, credit_sem):
    my = lax.axis_index("i")
    left = (my - 1) % N_DEV
    right = (my + 1) % N_DEV

    barrier = pltpu.get_barrier_semaphore()
    for nbr in (left, right):
        pl.semaphore_signal(
            barrier, inc=1, device_id=(nbr,),
            device_id_type=pl.DeviceIdType.MESH,
        )
    pl.semaphore_wait(barrier, 2)

    acc_ref[...] = p_ref[...]

    for k in range(N_DEV - 1):
        slot = k % 2
        s_idx = (my - k) % N_DEV
        r_idx = (my - k - 1) % N_DEV
        if k >= 2:
            pl.semaphore_wait(credit_sem.at[slot], 1)
        rdma = pltpu.make_async_remote_copy(
            src_ref=acc_ref.at[s_idx],
            dst_ref=comm_ref.at[slot],
            send_sem=send_sem.at[slot],
            recv_sem=recv_sem.at[slot],
            device_id=(right,),
            device_id_type=pl.DeviceIdType.MESH,
        )
        rdma.start()
        rdma.wait()
        acc_ref[r_idx] = acc_ref[r_idx] + comm_ref[slot]
        if k + 2 <= 2 * (N_DEV - 1) - 1:
            pl.semaphore_signal(
                credit_sem.at[slot], inc=1, device_id=(left,),
                device_id_type=pl.DeviceIdType.MESH,
            )

    own = (my + 1) % N_DEV
    out_ref[own] = acc_ref[own].astype(out_ref.dtype)

    for g in range(N_DEV - 1):
        t = (N_DEV - 1) + g
        slot = t % 2
        s_idx = (my + 1 - g) % N_DEV
        pl.semaphore_wait(credit_sem.at[slot], 1)
        rdma = pltpu.make_async_remote_copy(
            src_ref=out_ref.at[s_idx],
            dst_ref=out_ref.at[s_idx],
            send_sem=send_sem.at[slot],
            recv_sem=recv_sem.at[slot],
            device_id=(right,),
            device_id_type=pl.DeviceIdType.MESH,
        )
        rdma.start()
        rdma.wait()
        if t + 2 <= 2 * (N_DEV - 1) - 1:
            pl.semaphore_signal(
                credit_sem.at[slot], inc=1, device_id=(left,),
                device_id_type=pl.DeviceIdType.MESH,
            )


def _ring_allreduce(partial, cid):
    B, S, D = partial.shape
    rows = B * S
    c = rows // N_DEV
    p = partial.reshape(N_DEV, c, D)
    out = pl.pallas_call(
        _ar_body,
        out_shape=jax.ShapeDtypeStruct((N_DEV, c, D), jnp.bfloat16),
        in_specs=[pl.BlockSpec(memory_space=pltpu.VMEM)],
        out_specs=pl.BlockSpec(memory_space=pltpu.VMEM),
        scratch_shapes=[
            pltpu.VMEM((N_DEV, c, D), jnp.float32),
            pltpu.VMEM((2, c, D), jnp.float32),
            pltpu.SemaphoreType.DMA((2,)),
            pltpu.SemaphoreType.DMA((2,)),
            pltpu.SemaphoreType.REGULAR((2,)),
        ],
        compiler_params=pltpu.CompilerParams(collective_id=cid),
    )(p)
    return out.reshape(B, S, D)


def kernel(x, Wq, Wk, Wv, Wo, t_emb, W_mod, W_ff1, W_ff2):
    f32 = jnp.float32
    bf16 = jnp.bfloat16
    B, S, D = x.shape
    Dh = 128
    H = Wq.shape[1] // Dh

    mod = t_emb @ W_mod
    sa, sha, ga, sm, shm, gm = jnp.split(mod, 6, axis=-1)

    def ln(h):
        m = h.mean(axis=-1, keepdims=True)
        v = h.var(axis=-1, keepdims=True)
        return (h - m) * lax.rsqrt(v + 1e-5)

    x0 = x
    xm = (ln(x0) * (1.0 + sa[:, None, :]) + sha[:, None, :]).astype(bf16)

    Q = (xm @ Wq.astype(bf16)).reshape(B, S, H, Dh)
    K = (xm @ Wk.astype(bf16)).reshape(B, S, H, Dh)
    V = (xm @ Wv.astype(bf16)).reshape(B, S, H, Dh)
    scores = jnp.einsum(
        "bihd,bjhd->bhij", Q, K, preferred_element_type=f32
    ) * 0.08838834764831843
    P = jax.nn.softmax(scores, axis=-1).astype(bf16)
    O = jnp.einsum("bhij,bjhd->bihd", P, V, preferred_element_type=f32)
    attn_partial = O.reshape(B, S, H * Dh).astype(bf16) @ Wo.astype(bf16)
    attn_partial = attn_partial.astype(f32)

    attn_sum = _ring_allreduce(attn_partial, cid=0).astype(f32)
    x1 = x0 + ga[:, None, :] * attn_sum

    xm2 = (ln(x1) * (1.0 + sm[:, None, :]) + shm[:, None, :]).astype(bf16)
    h = (xm2 @ W_ff1.astype(bf16)).astype(f32)
    h = h * jax.nn.sigmoid(h)
    ff_partial = (h.astype(bf16) @ W_ff2.astype(bf16)).astype(f32)

    ff_sum = _ring_allreduce(ff_partial, cid=1).astype(f32)
    return x1 + gm[:, None, :] * ff_sum


# device time: 349622 ns/iter; 1.9238x vs baseline; 1.9238x over previous
import jax
import jax.numpy as jnp
from jax import lax
from jax.experimental import pallas as pl
from jax.experimental.pallas import tpu as pltpu

N_DEV = 32
G = 8
NZ = 4
LAST_T = 2 * (G - 1) - 1


def _ar_body(p_ref, out_ref, acc_ref, commg_ref, stage_ref, commz_ref,
             sendg_sem, recvg_sem, z_send_sem, z_recv_sem, credit_sem):
    my = lax.axis_index("i")
    p = my % G
    z = my // G
    right = z * G + (p + 1) % G
    left = z * G + (p - 1) % G
    zp1 = my ^ 8
    zp2 = my ^ 16

    barrier = pltpu.get_barrier_semaphore()
    for nbr in (left, right, zp1, zp2):
        pl.semaphore_signal(
            barrier, inc=1, device_id=(nbr,),
            device_id_type=pl.DeviceIdType.MESH,
        )
    pl.semaphore_wait(barrier, 4)

    acc_ref[...] = p_ref[...]

    for k in range(G - 1):
        slot = k % 2
        s_idx = (p - k) % G
        r_idx = (p - k - 1) % G
        if k >= 2:
            pl.semaphore_wait(credit_sem.at[slot], 1)
        stage_ref[slot] = acc_ref[s_idx].astype(stage_ref.dtype)
        rdma = pltpu.make_async_remote_copy(
            src_ref=stage_ref.at[slot],
            dst_ref=commg_ref.at[slot],
            send_sem=sendg_sem.at[slot],
            recv_sem=recvg_sem.at[slot],
            device_id=(right,),
            device_id_type=pl.DeviceIdType.MESH,
        )
        rdma.start()
        rdma.wait()
        acc_ref[r_idx] = acc_ref[r_idx] + commg_ref[slot].astype(jnp.float32)
        pl.semaphore_signal(
            credit_sem.at[slot], inc=1, device_id=(left,),
            device_id_type=pl.DeviceIdType.MESH,
        )

    o = (p + 1) % G
    zb0 = z % 2
    zb1 = z // 2
    hoff = zb0 * 128
    qoff = hoff + zb1 * 64

    stage_ref[0, 0:128] = acc_ref[o, pl.ds((1 - zb0) * 128, 128)].astype(
        stage_ref.dtype
    )
    rdma = pltpu.make_async_remote_copy(
        src_ref=stage_ref.at[0, pl.ds(0, 128)],
        dst_ref=commz_ref.at[0],
        send_sem=z_send_sem.at[0],
        recv_sem=z_recv_sem.at[0],
        device_id=(zp1,),
        device_id_type=pl.DeviceIdType.MESH,
    )
    rdma.start()
    rdma.wait()
    acc_ref[o, pl.ds(hoff, 128)] = (
        acc_ref[o, pl.ds(hoff, 128)] + commz_ref[0].astype(jnp.float32)
    )

    stage_ref[1, 0:64] = acc_ref[o, pl.ds(hoff + (1 - zb1) * 64, 64)].astype(
        stage_ref.dtype
    )
    rdma = pltpu.make_async_remote_copy(
        src_ref=stage_ref.at[1, pl.ds(0, 64)],
        dst_ref=commz_ref.at[1, pl.ds(0, 64)],
        send_sem=z_send_sem.at[1],
        recv_sem=z_recv_sem.at[1],
        device_id=(zp2,),
        device_id_type=pl.DeviceIdType.MESH,
    )
    rdma.start()
    rdma.wait()
    acc_ref[o, pl.ds(qoff, 64)] = (
        acc_ref[o, pl.ds(qoff, 64)] + commz_ref[1, 0:64].astype(jnp.float32)
    )

    out_ref[o, pl.ds(qoff, 64)] = acc_ref[o, pl.ds(qoff, 64)].astype(
        out_ref.dtype
    )

    rdma = pltpu.make_async_remote_copy(
        src_ref=out_ref.at[o, pl.ds(qoff, 64)],
        dst_ref=out_ref.at[o, pl.ds(qoff, 64)],
        send_sem=z_send_sem.at[2],
        recv_sem=z_recv_sem.at[2],
        device_id=(zp2,),
        device_id_type=pl.DeviceIdType.MESH,
    )
    rdma.start()
    rdma.wait()

    rdma = pltpu.make_async_remote_copy(
        src_ref=out_ref.at[o, pl.ds(hoff, 128)],
        dst_ref=out_ref.at[o, pl.ds(hoff, 128)],
        send_sem=z_send_sem.at[3],
        recv_sem=z_recv_sem.at[3],
        device_id=(zp1,),
        device_id_type=pl.DeviceIdType.MESH,
    )
    rdma.start()
    rdma.wait()

    for g in range(G - 1):
        t = (G - 1) + g
        slot = t % 2
        s_idx = (p + 1 - g) % G
        pl.semaphore_wait(credit_sem.at[slot], 1)
        rdma = pltpu.make_async_remote_copy(
            src_ref=out_ref.at[s_idx],
            dst_ref=out_ref.at[s_idx],
            send_sem=sendg_sem.at[slot],
            recv_sem=recvg_sem.at[slot],
            device_id=(right,),
            device_id_type=pl.DeviceIdType.MESH,
        )
        rdma.start()
        rdma.wait()
        if t + 2 <= LAST_T:
            pl.semaphore_signal(
                credit_sem.at[slot], inc=1, device_id=(left,),
                device_id_type=pl.DeviceIdType.MESH,
            )


def _ring_allreduce(partial, cid):
    B, S, D = partial.shape
    rows = B * S
    c = rows // G
    p = partial.reshape(G, c, D)
    out = pl.pallas_call(
        _ar_body,
        out_shape=jax.ShapeDtypeStruct((G, c, D), jnp.bfloat16),
        in_specs=[pl.BlockSpec(memory_space=pltpu.VMEM)],
        out_specs=pl.BlockSpec(memory_space=pltpu.VMEM),
        scratch_shapes=[
            pltpu.VMEM((G, c, D), jnp.float32),
            pltpu.VMEM((2, c, D), jnp.bfloat16),
            pltpu.VMEM((2, c, D), jnp.bfloat16),
            pltpu.VMEM((2, 128, D), jnp.bfloat16),
            pltpu.SemaphoreType.DMA((2,)),
            pltpu.SemaphoreType.DMA((2,)),
            pltpu.SemaphoreType.DMA((4,)),
            pltpu.SemaphoreType.DMA((4,)),
            pltpu.SemaphoreType.REGULAR((2,)),
        ],
        compiler_params=pltpu.CompilerParams(collective_id=cid),
    )(p)
    return out.reshape(B, S, D)


def kernel(x, Wq, Wk, Wv, Wo, t_emb, W_mod, W_ff1, W_ff2):
    f32 = jnp.float32
    bf16 = jnp.bfloat16
    B, S, D = x.shape
    Dh = 128
    H = Wq.shape[1] // Dh

    mod = t_emb @ W_mod
    sa, sha, ga, sm, shm, gm = jnp.split(mod, 6, axis=-1)

    def ln(h):
        m = h.mean(axis=-1, keepdims=True)
        v = h.var(axis=-1, keepdims=True)
        return (h - m) * lax.rsqrt(v + 1e-5)

    x0 = x
    xm = (ln(x0) * (1.0 + sa[:, None, :]) + sha[:, None, :]).astype(bf16)

    Q = (xm @ Wq.astype(bf16)).reshape(B, S, H, Dh)
    K = (xm @ Wk.astype(bf16)).reshape(B, S, H, Dh)
    V = (xm @ Wv.astype(bf16)).reshape(B, S, H, Dh)
    scores = jnp.einsum(
        "bihd,bjhd->bhij", Q, K, preferred_element_type=f32
    ) * 0.08838834764831843
    P = jax.nn.softmax(scores, axis=-1).astype(bf16)
    O = jnp.einsum("bhij,bjhd->bihd", P, V, preferred_element_type=f32)
    attn_partial = O.reshape(B, S, H * Dh).astype(bf16) @ Wo.astype(bf16)
    attn_partial = attn_partial.astype(f32)

    attn_sum = _ring_allreduce(attn_partial, cid=0).astype(f32)
    x1 = x0 + ga[:, None, :] * attn_sum

    xm2 = (ln(x1) * (1.0 + sm[:, None, :]) + shm[:, None, :]).astype(bf16)
    h = (xm2 @ W_ff1.astype(bf16)).astype(f32)
    h = h * jax.nn.sigmoid(h)
    ff_partial = (h.astype(bf16) @ W_ff2.astype(bf16)).astype(f32)

    ff_sum = _ring_allreduce(ff_partial, cid=1).astype(f32)
    return x1 + gm[:, None, :] * ff_sum


# device time: 332754 ns/iter; 2.0213x vs baseline; 1.0507x over previous
import jax
import jax.numpy as jnp
from jax import lax
from jax.experimental import pallas as pl
from jax.experimental.pallas import tpu as pltpu

N_DEV = 32
G = 8
HC = 128
LAST_T = 2 * (G - 1) - 1

_MESH = pl.DeviceIdType.MESH


def _ar_body(p_ref, out_ref, acc_ref, commg_ref, stage_ref, commz_ref,
             sendg_sem, recvg_sem, z_send_sem, z_recv_sem, credit_sem):
    my = lax.axis_index("i")
    p = my % G
    z = my // G
    right = z * G + (p + 1) % G
    left = z * G + (p - 1) % G
    zp1 = my ^ 8
    zp2 = my ^ 16

    barrier = pltpu.get_barrier_semaphore()
    for nbr in (left, right, zp1, zp2):
        pl.semaphore_signal(barrier, inc=1, device_id=(nbr,),
                            device_id_type=_MESH)
    pl.semaphore_wait(barrier, 4)

    acc_ref[...] = p_ref[...]
    bf = out_ref.dtype
    f32 = jnp.float32

    for k in range(G - 1):
        sR = k % 2
        sL = 2 + k % 2
        s_idxR = (p - k) % G
        r_idxR = (p - k - 1) % G
        s_idxL = (p + k) % G
        r_idxL = (p + k + 1) % G
        if k >= 2:
            pl.semaphore_wait(credit_sem.at[sR], 1)
            pl.semaphore_wait(credit_sem.at[sL], 1)
        stage_ref[sR] = acc_ref[s_idxR, 0:HC].astype(bf)
        stage_ref[sL] = acc_ref[s_idxL, HC:2 * HC].astype(bf)
        rdmaR = pltpu.make_async_remote_copy(
            src_ref=stage_ref.at[sR], dst_ref=commg_ref.at[sR],
            send_sem=sendg_sem.at[sR], recv_sem=recvg_sem.at[sR],
            device_id=(right,), device_id_type=_MESH,
        )
        rdmaL = pltpu.make_async_remote_copy(
            src_ref=stage_ref.at[sL], dst_ref=commg_ref.at[sL],
            send_sem=sendg_sem.at[sL], recv_sem=recvg_sem.at[sL],
            device_id=(left,), device_id_type=_MESH,
        )
        rdmaR.start()
        rdmaL.start()
        rdmaR.wait()
        rdmaL.wait()
        acc_ref[r_idxR, 0:HC] = acc_ref[r_idxR, 0:HC] + commg_ref[sR].astype(f32)
        acc_ref[r_idxL, HC:2 * HC] = (
            acc_ref[r_idxL, HC:2 * HC] + commg_ref[sL].astype(f32)
        )
        pl.semaphore_signal(credit_sem.at[sR], inc=1, device_id=(left,),
                            device_id_type=_MESH)
        pl.semaphore_signal(credit_sem.at[sL], inc=1, device_id=(right,),
                            device_id_type=_MESH)

    oR = (p + 1) % G
    oL = (p - 1) % G
    zb0 = z % 2
    zb1 = z // 2
    hR = zb0 * 64
    hL = HC + zb0 * 64
    qR = hR + zb1 * 32
    qL = hL + zb1 * 32

    stage_ref[0, 0:64] = acc_ref[oR, pl.ds((1 - zb0) * 64, 64)].astype(bf)
    stage_ref[2, 0:64] = acc_ref[oL, pl.ds(HC + (1 - zb0) * 64, 64)].astype(bf)
    ex0R = pltpu.make_async_remote_copy(
        src_ref=stage_ref.at[0, pl.ds(0, 64)], dst_ref=commz_ref.at[0],
        send_sem=z_send_sem.at[0], recv_sem=z_recv_sem.at[0],
        device_id=(zp1,), device_id_type=_MESH,
    )
    ex0L = pltpu.make_async_remote_copy(
        src_ref=stage_ref.at[2, pl.ds(0, 64)], dst_ref=commz_ref.at[1],
        send_sem=z_send_sem.at[1], recv_sem=z_recv_sem.at[1],
        device_id=(zp1,), device_id_type=_MESH,
    )
    ex0R.start()
    ex0L.start()
    ex0R.wait()
    ex0L.wait()
    acc_ref[oR, pl.ds(hR, 64)] = (
        acc_ref[oR, pl.ds(hR, 64)] + commz_ref[0].astype(f32)
    )
    acc_ref[oL, pl.ds(hL, 64)] = (
        acc_ref[oL, pl.ds(hL, 64)] + commz_ref[1].astype(f32)
    )

    stage_ref[1, 0:32] = acc_ref[oR, pl.ds(hR + (1 - zb1) * 32, 32)].astype(bf)
    stage_ref[3, 0:32] = acc_ref[oL, pl.ds(hL + (1 - zb1) * 32, 32)].astype(bf)
    ex1R = pltpu.make_async_remote_copy(
        src_ref=stage_ref.at[1, pl.ds(0, 32)],
        dst_ref=commz_ref.at[2, pl.ds(0, 32)],
        send_sem=z_send_sem.at[2], recv_sem=z_recv_sem.at[2],
        device_id=(zp2,), device_id_type=_MESH,
    )
    ex1L = pltpu.make_async_remote_copy(
        src_ref=stage_ref.at[3, pl.ds(0, 32)],
        dst_ref=commz_ref.at[3, pl.ds(0, 32)],
        send_sem=z_send_sem.at[3], recv_sem=z_recv_sem.at[3],
        device_id=(zp2,), device_id_type=_MESH,
    )
    ex1R.start()
    ex1L.start()
    ex1R.wait()
    ex1L.wait()
    acc_ref[oR, pl.ds(qR, 32)] = (
        acc_ref[oR, pl.ds(qR, 32)] + commz_ref[2, 0:32].astype(f32)
    )
    acc_ref[oL, pl.ds(qL, 32)] = (
        acc_ref[oL, pl.ds(qL, 32)] + commz_ref[3, 0:32].astype(f32)
    )

    out_ref[oR, pl.ds(qR, 32)] = acc_ref[oR, pl.ds(qR, 32)].astype(bf)
    out_ref[oL, pl.ds(qL, 32)] = acc_ref[oL, pl.ds(qL, 32)].astype(bf)

    ag2R = pltpu.make_async_remote_copy(
        src_ref=out_ref.at[oR, pl.ds(qR, 32)],
        dst_ref=out_ref.at[oR, pl.ds(qR, 32)],
        send_sem=z_send_sem.at[4], recv_sem=z_recv_sem.at[4],
        device_id=(zp2,), device_id_type=_MESH,
    )
    ag2L = pltpu.make_async_remote_copy(
        src_ref=out_ref.at[oL, pl.ds(qL, 32)],
        dst_ref=out_ref.at[oL, pl.ds(qL, 32)],
        send_sem=z_send_sem.at[5], recv_sem=z_recv_sem.at[5],
        device_id=(zp2,), device_id_type=_MESH,
    )
    ag2R.start()
    ag2L.start()
    ag2R.wait()
    ag2L.wait()

    ag3R = pltpu.make_async_remote_copy(
        src_ref=out_ref.at[oR, pl.ds(hR, 64)],
        dst_ref=out_ref.at[oR, pl.ds(hR, 64)],
        send_sem=z_send_sem.at[6], recv_sem=z_recv_sem.at[6],
        device_id=(zp1,), device_id_type=_MESH,
    )
    ag3L = pltpu.make_async_remote_copy(
        src_ref=out_ref.at[oL, pl.ds(hL, 64)],
        dst_ref=out_ref.at[oL, pl.ds(hL, 64)],
        send_sem=z_send_sem.at[7], recv_sem=z_recv_sem.at[7],
        device_id=(zp1,), device_id_type=_MESH,
    )
    ag3R.start()
    ag3L.start()
    ag3R.wait()
    ag3L.wait()

    for g in range(G - 1):
        t = (G - 1) + g
        sR = t % 2
        sL = 2 + t % 2
        s_idxR = (p + 1 - g) % G
        s_idxL = (p - 1 + g) % G
        pl.semaphore_wait(credit_sem.at[sR], 1)
        pl.semaphore_wait(credit_sem.at[sL], 1)
        agR = pltpu.make_async_remote_copy(
            src_ref=out_ref.at[s_idxR, pl.ds(0, HC)],
            dst_ref=out_ref.at[s_idxR, pl.ds(0, HC)],
            send_sem=sendg_sem.at[sR], recv_sem=recvg_sem.at[sR],
            device_id=(right,), device_id_type=_MESH,
        )
        agL = pltpu.make_async_remote_copy(
            src_ref=out_ref.at[s_idxL, pl.ds(HC, HC)],
            dst_ref=out_ref.at[s_idxL, pl.ds(HC, HC)],
            send_sem=sendg_sem.at[sL], recv_sem=recvg_sem.at[sL],
            device_id=(left,), device_id_type=_MESH,
        )
        agR.start()
        agL.start()
        agR.wait()
        agL.wait()
        if t + 2 <= LAST_T:
            pl.semaphore_signal(credit_sem.at[sR], inc=1, device_id=(left,),
                                device_id_type=_MESH)
            pl.semaphore_signal(credit_sem.at[sL], inc=1, device_id=(right,),
                                device_id_type=_MESH)


def _ring_allreduce(partial, cid):
    B, S, D = partial.shape
    rows = B * S
    c = rows // G
    p = partial.reshape(G, c, D)
    out = pl.pallas_call(
        _ar_body,
        out_shape=jax.ShapeDtypeStruct((G, c, D), jnp.bfloat16),
        in_specs=[pl.BlockSpec(memory_space=pltpu.VMEM)],
        out_specs=pl.BlockSpec(memory_space=pltpu.VMEM),
        scratch_shapes=[
            pltpu.VMEM((G, c, D), jnp.float32),
            pltpu.VMEM((4, HC, D), jnp.bfloat16),
            pltpu.VMEM((4, HC, D), jnp.bfloat16),
            pltpu.VMEM((4, 64, D), jnp.bfloat16),
            pltpu.SemaphoreType.DMA((4,)),
            pltpu.SemaphoreType.DMA((4,)),
            pltpu.SemaphoreType.DMA((8,)),
            pltpu.SemaphoreType.DMA((8,)),
            pltpu.SemaphoreType.REGULAR((4,)),
        ],
        compiler_params=pltpu.CompilerParams(collective_id=cid),
    )(p)
    return out.reshape(B, S, D)


def kernel(x, Wq, Wk, Wv, Wo, t_emb, W_mod, W_ff1, W_ff2):
    f32 = jnp.float32
    bf16 = jnp.bfloat16
    B, S, D = x.shape
    Dh = 128
    H = Wq.shape[1] // Dh

    mod = t_emb @ W_mod
    sa, sha, ga, sm, shm, gm = jnp.split(mod, 6, axis=-1)

    def ln(h):
        m = h.mean(axis=-1, keepdims=True)
        v = h.var(axis=-1, keepdims=True)
        return (h - m) * lax.rsqrt(v + 1e-5)

    x0 = x
    xm = (ln(x0) * (1.0 + sa[:, None, :]) + sha[:, None, :]).astype(bf16)

    Q = (xm @ Wq.astype(bf16)).reshape(B, S, H, Dh)
    K = (xm @ Wk.astype(bf16)).reshape(B, S, H, Dh)
    V = (xm @ Wv.astype(bf16)).reshape(B, S, H, Dh)
    scores = jnp.einsum(
        "bihd,bjhd->bhij", Q, K, preferred_element_type=f32
    ) * 0.08838834764831843
    P = jax.nn.softmax(scores, axis=-1).astype(bf16)
    O = jnp.einsum("bhij,bjhd->bihd", P, V, preferred_element_type=f32)
    attn_partial = O.reshape(B, S, H * Dh).astype(bf16) @ Wo.astype(bf16)
    attn_partial = attn_partial.astype(f32)

    attn_sum = _ring_allreduce(attn_partial, cid=0).astype(f32)
    x1 = x0 + ga[:, None, :] * attn_sum

    xm2 = (ln(x1) * (1.0 + sm[:, None, :]) + shm[:, None, :]).astype(bf16)
    h = (xm2 @ W_ff1.astype(bf16)).astype(f32)
    h = h * jax.nn.sigmoid(h)
    ff_partial = (h.astype(bf16) @ W_ff2.astype(bf16)).astype(f32)

    ff_sum = _ring_allreduce(ff_partial, cid=1).astype(f32)
    return x1 + gm[:, None, :] * ff_sum


# device time: 332187 ns/iter; 2.0248x vs baseline; 1.0017x over previous
import jax
import jax.numpy as jnp
from jax import lax
from jax.experimental import pallas as pl
from jax.experimental.pallas import tpu as pltpu

G = 8
HC = 128
LAST_T = 2 * (G - 1) - 1

_MESH = pl.DeviceIdType.MESH


def _ar_body(p_ref, out_ref, acc_ref, commg_ref, stage_ref, commz_ref,
             sendg_sem, recvg_sem, z_send_sem, z_recv_sem, credit_sem):
    my = lax.axis_index("i")
    p = my % G
    z = my // G
    right = z * G + (p + 1) % G
    left = z * G + (p - 1) % G
    zp1 = my ^ 8
    zp2 = my ^ 16

    barrier = pltpu.get_barrier_semaphore()
    for nbr in (left, right, zp1, zp2):
        pl.semaphore_signal(barrier, inc=1, device_id=(nbr,),
                            device_id_type=_MESH)
    pl.semaphore_wait(barrier, 4)

    bf = out_ref.dtype
    f32 = jnp.float32

    def ring_rdma(slot, src, dst, dev):
        return pltpu.make_async_remote_copy(
            src_ref=src, dst_ref=dst,
            send_sem=sendg_sem.at[slot], recv_sem=recvg_sem.at[slot],
            device_id=(dev,), device_id_type=_MESH,
        )

    def z_rdma(i, src, dst, dev):
        return pltpu.make_async_remote_copy(
            src_ref=src, dst_ref=dst,
            send_sem=z_send_sem.at[i], recv_sem=z_recv_sem.at[i],
            device_id=(dev,), device_id_type=_MESH,
        )

    def credit_to(slot, dev):
        pl.semaphore_signal(credit_sem.at[slot], inc=1, device_id=(dev,),
                            device_id_type=_MESH)

    rs = []
    for k in range(G - 1):
        sR = k % 2
        sL = 2 + k % 2
        s_idxR = (p - k) % G
        s_idxL = (p + k) % G
        if k >= 2:
            pl.semaphore_wait(credit_sem.at[sR], 1)
            pl.semaphore_wait(credit_sem.at[sL], 1)
            rs[k - 2][0].wait_send()
            rs[k - 2][1].wait_send()
        if k == 0:
            stage_ref[sR] = p_ref[s_idxR, 0:HC].astype(bf)
            stage_ref[sL] = p_ref[s_idxL, HC:2 * HC].astype(bf)
        else:
            pR = (k - 1) % 2
            pL = 2 + (k - 1) % 2
            stage_ref[sR] = (
                p_ref[s_idxR, 0:HC] + commg_ref[pR].astype(f32)
            ).astype(bf)
            stage_ref[sL] = (
                p_ref[s_idxL, HC:2 * HC] + commg_ref[pL].astype(f32)
            ).astype(bf)
            credit_to(pR, left)
            credit_to(pL, right)
        rdmaR = ring_rdma(sR, stage_ref.at[sR], commg_ref.at[sR], right)
        rdmaL = ring_rdma(sL, stage_ref.at[sL], commg_ref.at[sL], left)
        rdmaR.start()
        rdmaL.start()
        rs.append((rdmaR, rdmaL))
        rdmaR.wait_recv()
        rdmaL.wait_recv()

    oR = (p + 1) % G
    oL = (p - 1) % G
    acc_ref[0] = p_ref[oR, 0:HC] + commg_ref[0].astype(f32)
    acc_ref[1] = p_ref[oL, HC:2 * HC] + commg_ref[2].astype(f32)
    credit_to(0, left)
    credit_to(2, right)

    zb0 = z % 2
    zb1 = z // 2
    hh = zb0 * 64
    qq = hh + zb1 * 32
    sq = hh + (1 - zb1) * 32
    qk = zb1 * 32
    qs = (1 - zb1) * 32

    pend = []

    rs[6][0].wait_send()
    rs[6][1].wait_send()
    stage_ref[0, 0:64] = acc_ref[0, pl.ds((1 - zb0) * 64, 64)].astype(bf)
    stage_ref[2, 0:64] = acc_ref[1, pl.ds((1 - zb0) * 64, 64)].astype(bf)
    ex0R = z_rdma(0, stage_ref.at[0, pl.ds(0, 64)], commz_ref.at[0], zp1)
    ex0L = z_rdma(1, stage_ref.at[2, pl.ds(0, 64)], commz_ref.at[1], zp1)
    ex0R.start()
    ex0L.start()
    ex0R.wait_recv()
    ex0L.wait_recv()
    pend += [ex0R, ex0L]

    rs[5][0].wait_send()
    rs[5][1].wait_send()
    stage_ref[1, 0:32] = (
        acc_ref[0, pl.ds(sq, 32)] + commz_ref[0, pl.ds(qs, 32)].astype(f32)
    ).astype(bf)
    stage_ref[3, 0:32] = (
        acc_ref[1, pl.ds(sq, 32)] + commz_ref[1, pl.ds(qs, 32)].astype(f32)
    ).astype(bf)
    ex1R = z_rdma(2, stage_ref.at[1, pl.ds(0, 32)],
                  commz_ref.at[2, pl.ds(0, 32)], zp2)
    ex1L = z_rdma(3, stage_ref.at[3, pl.ds(0, 32)],
                  commz_ref.at[3, pl.ds(0, 32)], zp2)
    ex1R.start()
    ex1L.start()
    acc_ref[0, pl.ds(qq, 32)] = (
        acc_ref[0, pl.ds(qq, 32)] + commz_ref[0, pl.ds(qk, 32)].astype(f32)
    )
    acc_ref[1, pl.ds(qq, 32)] = (
        acc_ref[1, pl.ds(qq, 32)] + commz_ref[1, pl.ds(qk, 32)].astype(f32)
    )
    ex1R.wait_recv()
    ex1L.wait_recv()
    out_ref[oR, pl.ds(qq, 32)] = (
        acc_ref[0, pl.ds(qq, 32)] + commz_ref[2, pl.ds(0, 32)].astype(f32)
    ).astype(bf)
    out_ref[oL, pl.ds(HC + qq, 32)] = (
        acc_ref[1, pl.ds(qq, 32)] + commz_ref[3, pl.ds(0, 32)].astype(f32)
    ).astype(bf)
    pend += [ex1R, ex1L]

    ag2R = z_rdma(4, out_ref.at[oR, pl.ds(qq, 32)],
                  out_ref.at[oR, pl.ds(qq, 32)], zp2)
    ag2L = z_rdma(5, out_ref.at[oL, pl.ds(HC + qq, 32)],
                  out_ref.at[oL, pl.ds(HC + qq, 32)], zp2)
    ag2R.start()
    ag2L.start()
    ag2R.wait_recv()
    ag2L.wait_recv()
    pend += [ag2R, ag2L]

    ag3R = z_rdma(6, out_ref.at[oR, pl.ds(hh, 64)],
                  out_ref.at[oR, pl.ds(hh, 64)], zp1)
    ag3L = z_rdma(7, out_ref.at[oL, pl.ds(HC + hh, 64)],
                  out_ref.at[oL, pl.ds(HC + hh, 64)], zp1)
    ag3R.start()
    ag3L.start()
    ag3R.wait_recv()
    ag3L.wait_recv()
    pend += [ag3R, ag3L]

    ag = []
    for g in range(G - 1):
        t = (G - 1) + g
        sR = t % 2
        sL = 2 + t % 2
        s_idxR = (p + 1 - g) % G
        s_idxL = (p - 1 + g) % G
        pl.semaphore_wait(credit_sem.at[sR], 1)
        pl.semaphore_wait(credit_sem.at[sL], 1)
        if g >= 1:
            ag[g - 1][0].wait_recv()
            ag[g - 1][1].wait_recv()
            if g <= 5:
                credit_to((t - 1) % 2, left)
                credit_to(2 + (t - 1) % 2, right)
        agR = ring_rdma(sR, out_ref.at[s_idxR, pl.ds(0, HC)],
                        out_ref.at[s_idxR, pl.ds(0, HC)], right)
        agL = ring_rdma(sL, out_ref.at[s_idxL, pl.ds(HC, HC)],
                        out_ref.at[s_idxL, pl.ds(HC, HC)], left)
        agR.start()
        agL.start()
        ag.append((agR, agL))
    ag[6][0].wait_recv()
    ag[6][1].wait_recv()

    for r in pend:
        r.wait_send()
    for pair in ag:
        pair[0].wait_send()
        pair[1].wait_send()


def _ring_allreduce(partial, cid):
    B, S, D = partial.shape
    rows = B * S
    c = rows // G
    p = partial.reshape(G, c, D)
    out = pl.pallas_call(
        _ar_body,
        out_shape=jax.ShapeDtypeStruct((G, c, D), jnp.bfloat16),
        in_specs=[pl.BlockSpec(memory_space=pltpu.VMEM)],
        out_specs=pl.BlockSpec(memory_space=pltpu.VMEM),
        scratch_shapes=[
            pltpu.VMEM((2, HC, D), jnp.float32),
            pltpu.VMEM((4, HC, D), jnp.bfloat16),
            pltpu.VMEM((4, HC, D), jnp.bfloat16),
            pltpu.VMEM((4, 64, D), jnp.bfloat16),
            pltpu.SemaphoreType.DMA((4,)),
            pltpu.SemaphoreType.DMA((4,)),
            pltpu.SemaphoreType.DMA((8,)),
            pltpu.SemaphoreType.DMA((8,)),
            pltpu.SemaphoreType.REGULAR((4,)),
        ],
        compiler_params=pltpu.CompilerParams(collective_id=cid),
    )(p)
    return out.reshape(B, S, D)


def kernel(x, Wq, Wk, Wv, Wo, t_emb, W_mod, W_ff1, W_ff2):
    f32 = jnp.float32
    bf16 = jnp.bfloat16
    B, S, D = x.shape
    Dh = 128
    H = Wq.shape[1] // Dh

    mod = t_emb @ W_mod
    sa, sha, ga, sm, shm, gm = jnp.split(mod, 6, axis=-1)

    def ln(h):
        m = h.mean(axis=-1, keepdims=True)
        v = h.var(axis=-1, keepdims=True)
        return (h - m) * lax.rsqrt(v + 1e-5)

    x0 = x
    xm = (ln(x0) * (1.0 + sa[:, None, :]) + sha[:, None, :]).astype(bf16)

    Q = (xm @ Wq.astype(bf16)).reshape(B, S, H, Dh)
    K = (xm @ Wk.astype(bf16)).reshape(B, S, H, Dh)
    V = (xm @ Wv.astype(bf16)).reshape(B, S, H, Dh)
    scores = jnp.einsum(
        "bihd,bjhd->bhij", Q, K, preferred_element_type=f32
    ) * 0.08838834764831843
    P = jax.nn.softmax(scores, axis=-1).astype(bf16)
    O = jnp.einsum("bhij,bjhd->bihd", P, V, preferred_element_type=f32)
    attn_partial = O.reshape(B, S, H * Dh).astype(bf16) @ Wo.astype(bf16)
    attn_partial = attn_partial.astype(f32)

    attn_sum = _ring_allreduce(attn_partial, cid=0).astype(f32)
    x1 = x0 + ga[:, None, :] * attn_sum

    xm2 = (ln(x1) * (1.0 + sm[:, None, :]) + shm[:, None, :]).astype(bf16)
    h = (xm2 @ W_ff1.astype(bf16)).astype(f32)
    h = h * jax.nn.sigmoid(h)
    ff_partial = (h.astype(bf16) @ W_ff2.astype(bf16)).astype(f32)

    ff_sum = _ring_allreduce(ff_partial, cid=1).astype(f32)
    return x1 + gm[:, None, :] * ff_sum


# device time: 266958 ns/iter; 2.5195x vs baseline; 1.2443x over previous
import jax
import jax.numpy as jnp
from jax import lax
from jax.experimental import pallas as pl
from jax.experimental.pallas import tpu as pltpu

G = 8
HC = 128
LAST_T = 2 * (G - 1) - 1

_MESH = pl.DeviceIdType.MESH

_INV = (0, 7, 6, 1, 2, 5, 4, 3)
_NEXT = (3, 0, 1, 4, 7, 2, 5, 6)
_PREV = (1, 2, 5, 0, 3, 6, 7, 4)


def _lut(idx, table):
    v = jnp.int32(table[7])
    for i in range(6, -1, -1):
        v = jnp.where(idx == i, jnp.int32(table[i]), v)
    return v


def _ar_body(p_ref, out_ref, acc_ref, commg_ref, stage_ref, commz_ref,
             sendg_sem, recvg_sem, z_send_sem, z_recv_sem, credit_sem):
    my = lax.axis_index("i")
    p = my % G
    z = my // G
    r = _lut(p, _INV)
    right = z * G + _lut(p, _NEXT)
    left = z * G + _lut(p, _PREV)
    zp1 = my ^ 8
    zp2 = my ^ 16

    barrier = pltpu.get_barrier_semaphore()
    for nbr in (left, right, zp1, zp2):
        pl.semaphore_signal(barrier, inc=1, device_id=(nbr,),
                            device_id_type=_MESH)
    pl.semaphore_wait(barrier, 4)

    bf = out_ref.dtype
    f32 = jnp.float32

    def ring_rdma(slot, src, dst, dev):
        return pltpu.make_async_remote_copy(
            src_ref=src, dst_ref=dst,
            send_sem=sendg_sem.at[slot], recv_sem=recvg_sem.at[slot],
            device_id=(dev,), device_id_type=_MESH,
        )

    def z_rdma(i, src, dst, dev):
        return pltpu.make_async_remote_copy(
            src_ref=src, dst_ref=dst,
            send_sem=z_send_sem.at[i], recv_sem=z_recv_sem.at[i],
            device_id=(dev,), device_id_type=_MESH,
        )

    def credit_to(slot, dev):
        pl.semaphore_signal(credit_sem.at[slot], inc=1, device_id=(dev,),
                            device_id_type=_MESH)

    rs = []
    for k in range(G - 1):
        sR = k % 2
        sL = 2 + k % 2
        s_idxR = (r - k) % G
        s_idxL = (r + k) % G
        if k >= 2:
            pl.semaphore_wait(credit_sem.at[sR], 1)
            pl.semaphore_wait(credit_sem.at[sL], 1)
            rs[k - 2][0].wait_send()
            rs[k - 2][1].wait_send()
        if k == 0:
            stage_ref[sR] = p_ref[s_idxR, 0:HC].astype(bf)
            stage_ref[sL] = p_ref[s_idxL, HC:2 * HC].astype(bf)
        else:
            pR = (k - 1) % 2
            pL = 2 + (k - 1) % 2
            stage_ref[sR] = (
                p_ref[s_idxR, 0:HC] + commg_ref[pR].astype(f32)
            ).astype(bf)
            stage_ref[sL] = (
                p_ref[s_idxL, HC:2 * HC] + commg_ref[pL].astype(f32)
            ).astype(bf)
            credit_to(pR, left)
            credit_to(pL, right)
        rdmaR = ring_rdma(sR, stage_ref.at[sR], commg_ref.at[sR], right)
        rdmaL = ring_rdma(sL, stage_ref.at[sL], commg_ref.at[sL], left)
        rdmaR.start()
        rdmaL.start()
        rs.append((rdmaR, rdmaL))
        rdmaR.wait_recv()
        rdmaL.wait_recv()

    oR = (r + 1) % G
    oL = (r - 1) % G
    acc_ref[0] = p_ref[oR, 0:HC] + commg_ref[0].astype(f32)
    acc_ref[1] = p_ref[oL, HC:2 * HC] + commg_ref[2].astype(f32)
    credit_to(0, left)
    credit_to(2, right)

    zb0 = z % 2
    zb1 = z // 2
    hh = zb0 * 64
    qq = hh + zb1 * 32
    sq = hh + (1 - zb1) * 32
    qk = zb1 * 32
    qs = (1 - zb1) * 32

    pend = []

    rs[6][0].wait_send()
    rs[6][1].wait_send()
    stage_ref[0, 0:64] = acc_ref[0, pl.ds((1 - zb0) * 64, 64)].astype(bf)
    stage_ref[2, 0:64] = acc_ref[1, pl.ds((1 - zb0) * 64, 64)].astype(bf)
    ex0R = z_rdma(0, stage_ref.at[0, pl.ds(0, 64)], commz_ref.at[0], zp1)
    ex0L = z_rdma(1, stage_ref.at[2, pl.ds(0, 64)], commz_ref.at[1], zp1)
    ex0R.start()
    ex0L.start()
    ex0R.wait_recv()
    ex0L.wait_recv()
    pend += [ex0R, ex0L]

    rs[5][0].wait_send()
    rs[5][1].wait_send()
    stage_ref[1, 0:32] = (
        acc_ref[0, pl.ds(sq, 32)] + commz_ref[0, pl.ds(qs, 32)].astype(f32)
    ).astype(bf)
    stage_ref[3, 0:32] = (
        acc_ref[1, pl.ds(sq, 32)] + commz_ref[1, pl.ds(qs, 32)].astype(f32)
    ).astype(bf)
    ex1R = z_rdma(2, stage_ref.at[1, pl.ds(0, 32)],
                  commz_ref.at[2, pl.ds(0, 32)], zp2)
    ex1L = z_rdma(3, stage_ref.at[3, pl.ds(0, 32)],
                  commz_ref.at[3, pl.ds(0, 32)], zp2)
    ex1R.start()
    ex1L.start()
    acc_ref[0, pl.ds(qq, 32)] = (
        acc_ref[0, pl.ds(qq, 32)] + commz_ref[0, pl.ds(qk, 32)].astype(f32)
    )
    acc_ref[1, pl.ds(qq, 32)] = (
        acc_ref[1, pl.ds(qq, 32)] + commz_ref[1, pl.ds(qk, 32)].astype(f32)
    )
    ex1R.wait_recv()
    ex1L.wait_recv()
    out_ref[oR, pl.ds(qq, 32)] = (
        acc_ref[0, pl.ds(qq, 32)] + commz_ref[2, pl.ds(0, 32)].astype(f32)
    ).astype(bf)
    out_ref[oL, pl.ds(HC + qq, 32)] = (
        acc_ref[1, pl.ds(qq, 32)] + commz_ref[3, pl.ds(0, 32)].astype(f32)
    ).astype(bf)
    pend += [ex1R, ex1L]

    ag2R = z_rdma(4, out_ref.at[oR, pl.ds(qq, 32)],
                  out_ref.at[oR, pl.ds(qq, 32)], zp2)
    ag2L = z_rdma(5, out_ref.at[oL, pl.ds(HC + qq, 32)],
                  out_ref.at[oL, pl.ds(HC + qq, 32)], zp2)
    ag2R.start()
    ag2L.start()
    ag2R.wait_recv()
    ag2L.wait_recv()
    pend += [ag2R, ag2L]

    ag3R = z_rdma(6, out_ref.at[oR, pl.ds(hh, 64)],
                  out_ref.at[oR, pl.ds(hh, 64)], zp1)
    ag3L = z_rdma(7, out_ref.at[oL, pl.ds(HC + hh, 64)],
                  out_ref.at[oL, pl.ds(HC + hh, 64)], zp1)
    ag3R.start()
    ag3L.start()
    ag3R.wait_recv()
    ag3L.wait_recv()
    pend += [ag3R, ag3L]

    ag = []
    for g in range(G - 1):
        t = (G - 1) + g
        sR = t % 2
        sL = 2 + t % 2
        s_idxR = (r + 1 - g) % G
        s_idxL = (r - 1 + g) % G
        pl.semaphore_wait(credit_sem.at[sR], 1)
        pl.semaphore_wait(credit_sem.at[sL], 1)
        if g >= 1:
            ag[g - 1][0].wait_recv()
            ag[g - 1][1].wait_recv()
            if g <= 5:
                credit_to((t - 1) % 2, left)
                credit_to(2 + (t - 1) % 2, right)
        agR = ring_rdma(sR, out_ref.at[s_idxR, pl.ds(0, HC)],
                        out_ref.at[s_idxR, pl.ds(0, HC)], right)
        agL = ring_rdma(sL, out_ref.at[s_idxL, pl.ds(HC, HC)],
                        out_ref.at[s_idxL, pl.ds(HC, HC)], left)
        agR.start()
        agL.start()
        ag.append((agR, agL))
    ag[6][0].wait_recv()
    ag[6][1].wait_recv()

    for r in pend:
        r.wait_send()
    for pair in ag:
        pair[0].wait_send()
        pair[1].wait_send()


def _ring_allreduce(partial, cid):
    B, S, D = partial.shape
    rows = B * S
    c = rows // G
    p = partial.reshape(G, c, D)
    out = pl.pallas_call(
        _ar_body,
        out_shape=jax.ShapeDtypeStruct((G, c, D), jnp.bfloat16),
        in_specs=[pl.BlockSpec(memory_space=pltpu.VMEM)],
        out_specs=pl.BlockSpec(memory_space=pltpu.VMEM),
        scratch_shapes=[
            pltpu.VMEM((2, HC, D), jnp.float32),
            pltpu.VMEM((4, HC, D), jnp.bfloat16),
            pltpu.VMEM((4, HC, D), jnp.bfloat16),
            pltpu.VMEM((4, 64, D), jnp.bfloat16),
            pltpu.SemaphoreType.DMA((4,)),
            pltpu.SemaphoreType.DMA((4,)),
            pltpu.SemaphoreType.DMA((8,)),
            pltpu.SemaphoreType.DMA((8,)),
            pltpu.SemaphoreType.REGULAR((4,)),
        ],
        compiler_params=pltpu.CompilerParams(collective_id=cid),
    )(p)
    return out.reshape(B, S, D)


def kernel(x, Wq, Wk, Wv, Wo, t_emb, W_mod, W_ff1, W_ff2):
    f32 = jnp.float32
    bf16 = jnp.bfloat16
    B, S, D = x.shape
    Dh = 128
    H = Wq.shape[1] // Dh

    mod = t_emb @ W_mod
    sa, sha, ga, sm, shm, gm = jnp.split(mod, 6, axis=-1)

    def ln(h):
        m = h.mean(axis=-1, keepdims=True)
        v = h.var(axis=-1, keepdims=True)
        return (h - m) * lax.rsqrt(v + 1e-5)

    x0 = x
    xm = (ln(x0) * (1.0 + sa[:, None, :]) + sha[:, None, :]).astype(bf16)

    Q = (xm @ Wq.astype(bf16)).reshape(B, S, H, Dh)
    K = (xm @ Wk.astype(bf16)).reshape(B, S, H, Dh)
    V = (xm @ Wv.astype(bf16)).reshape(B, S, H, Dh)
    scores = jnp.einsum(
        "bihd,bjhd->bhij", Q, K, preferred_element_type=f32
    ) * 0.08838834764831843
    P = jax.nn.softmax(scores, axis=-1).astype(bf16)
    O = jnp.einsum("bhij,bjhd->bihd", P, V, preferred_element_type=f32)
    attn_partial = O.reshape(B, S, H * Dh).astype(bf16) @ Wo.astype(bf16)
    attn_partial = attn_partial.astype(f32)

    attn_sum = _ring_allreduce(attn_partial, cid=0).astype(f32)
    x1 = x0 + ga[:, None, :] * attn_sum

    xm2 = (ln(x1) * (1.0 + sm[:, None, :]) + shm[:, None, :]).astype(bf16)
    h = (xm2 @ W_ff1.astype(bf16)).astype(f32)
    h = h * jax.nn.sigmoid(h)
    ff_partial = (h.astype(bf16) @ W_ff2.astype(bf16)).astype(f32)

    ff_sum = _ring_allreduce(ff_partial, cid=1).astype(f32)
    return x1 + gm[:, None, :] * ff_sum
